# baseline (speedup 1.0000x reference)
import functools

import jax
import jax.numpy as jnp
from jax import lax
from jax.experimental import pallas as pl
from jax.experimental.pallas import tpu as pltpu
from jax.experimental.pallas import tpu_sc as plsc

_V = 1024
_D = 128
_W = _D // 2 + 1
_BR = 64
_SW = _D + 1


@functools.cache
def _build(n_total: int, nw: int):
    per_w = n_total // nw
    nblk = per_w // _BR
    mesh = plsc.VectorSubcoreMesh(core_axis_name="c", subcore_axis_name="s")

    @functools.partial(
        pl.kernel,
        mesh=mesh,
        compiler_params=pltpu.CompilerParams(
            needs_layout_passes=False, disable_bounds_checks=True,
            use_tc_tiling_on_sc=False),
        out_type=jax.ShapeDtypeStruct((n_total, _D), jnp.float32),
        scratch_types=[
            pltpu.VMEM((_V, _W), jnp.int32),
            pltpu.VMEM((2, _BR), jnp.int32),
            pltpu.VMEM((2 * _BR, _SW), jnp.float32),
            pltpu.SemaphoreType.DMA,
            pltpu.SemaphoreType.DMA,
        ],
    )
    def gather_kernel(ids_hbm, cb_hbm, out_hbm, cb_v, idx_v, stg_v, isem, osem):
        cid = lax.axis_index("c")
        sid = lax.axis_index("s")
        wid = sid * (nw // 16) + cid

        pltpu.sync_copy(cb_hbm, cb_v)

        base = wid * per_w
        iota16 = lax.iota(jnp.int32, 16)

        def idx_start(blk, b):
            return pltpu.async_copy(ids_hbm.at[wid, blk], idx_v.at[b], isem)

        def idx_wait(blk, b):
            pltpu.make_async_copy(ids_hbm.at[wid, blk], idx_v.at[b], isem).wait()

        def out_desc(blk, b):
            return pltpu.make_async_copy(
                stg_v.at[pl.ds(b * _BR, _BR), pl.ds(0, _D)],
                out_hbm.at[pl.ds(base + blk * _BR, _BR)],
                osem)

        idx_start(0, 0)
        idx_start(1, 1)

        def body(g, carry):
            for b in range(2):
                blk = 2 * g + b
                idx_wait(blk, b)

                @pl.when(blk >= 2)
                def _():
                    out_desc(blk - 2, b).wait()

                idxb = idx_v.at[b]

                def grp(gg, c2):
                    ivec = idxb[pl.ds(gg * 16, 16)]
                    rowv = (b * _BR + gg * 16) + iota16
                    col = jnp.zeros((16,), jnp.int32)
                    for _ in range(_D // 2):
                        x = plsc.load_gather(cb_v, [ivec, col])
                        lo = plsc.bitcast(lax.shift_left(x, 16), jnp.float32)
                        hi = plsc.bitcast(
                            lax.bitwise_and(x, jnp.int32(-65536)), jnp.float32)
                        plsc.store_scatter(stg_v, [rowv, col], lo)
                        plsc.store_scatter(stg_v, [rowv, col + (_D // 2)], hi)
                        col = col + 1
                    return c2

                lax.fori_loop(0, _BR // 16, grp, 0)

                out_desc(blk, b).start()

                @pl.when(blk + 2 < nblk)
                def _():
                    idx_start(blk + 2, b)
            return carry

        lax.fori_loop(0, nblk // 2, body, 0)
        out_desc(nblk - 2, 0).wait()
        out_desc(nblk - 1, 1).wait()

    return gather_kernel


def kernel(code_ids, codebook):
    b, k = code_ids.shape
    n = b * k
    info = plsc.get_sparse_core_info()
    nw = info.num_cores * info.num_subcores
    per_w = n // nw
    assert n % nw == 0 and per_w % _BR == 0 and (per_w // _BR) % 2 == 0, (n, nw)
    ids = code_ids.reshape(nw, per_w // _BR, _BR).astype(jnp.int32)
    cb_bf = codebook.astype(jnp.bfloat16)
    cb_pk = lax.bitcast_convert_type(
        jnp.stack([cb_bf[:, : _D // 2], cb_bf[:, _D // 2:]], axis=-1), jnp.int32)
    cb_pk = jnp.pad(cb_pk, ((0, 0), (0, _W - _D // 2)))
    out = _build(n, nw)(ids, cb_pk)
    return out.reshape(b, k, _D)


# speedup vs baseline: 1.5124x; 1.5124x over previous
import functools

import jax
import jax.numpy as jnp
from jax import lax
from jax.experimental import pallas as pl
from jax.experimental.pallas import tpu as pltpu
from jax.experimental.pallas import tpu_sc as plsc

_V = 1024
_D = 128
_W = _D // 2 + 1
_BR = 64
_SW = _D + 1


@functools.cache
def _build(n_total: int, nw: int):
    per_w = n_total // nw
    nblk = per_w // _BR
    mesh = plsc.VectorSubcoreMesh(core_axis_name="c", subcore_axis_name="s")

    @functools.partial(
        pl.kernel,
        mesh=mesh,
        compiler_params=pltpu.CompilerParams(
            needs_layout_passes=False, disable_bounds_checks=True,
            use_tc_tiling_on_sc=False),
        out_type=jax.ShapeDtypeStruct((n_total, _D), jnp.float32),
        scratch_types=[
            pltpu.VMEM((_V, _W), jnp.int32),
            pltpu.VMEM((2, _BR), jnp.int32),
            pltpu.VMEM((2 * _BR, _SW), jnp.float32),
            pltpu.SemaphoreType.DMA,
            pltpu.SemaphoreType.DMA,
        ],
    )
    def gather_kernel(ids_hbm, cb_hbm, out_hbm, cb_v, idx_v, stg_v, isem, osem):
        cid = lax.axis_index("c")
        sid = lax.axis_index("s")
        wid = sid * (nw // 16) + cid

        pltpu.sync_copy(cb_hbm, cb_v)

        base = wid * per_w
        iota16 = lax.iota(jnp.int32, 16)

        def idx_start(blk, b):
            return pltpu.async_copy(ids_hbm.at[wid, blk], idx_v.at[b], isem)

        def idx_wait(blk, b):
            pltpu.make_async_copy(ids_hbm.at[wid, blk], idx_v.at[b], isem).wait()

        def out_desc(blk, b):
            return pltpu.make_async_copy(
                stg_v.at[pl.ds(b * _BR, _BR), pl.ds(0, _D)],
                out_hbm.at[pl.ds(base + blk * _BR, _BR)],
                osem)

        idx_start(0, 0)
        idx_start(1, 1)

        def body(g, carry):
            for b in range(2):
                blk = 2 * g + b
                idx_wait(blk, b)

                @pl.when(blk >= 2)
                def _():
                    out_desc(blk - 2, b).wait()

                idxb = idx_v.at[b]

                def grp(gg, c2):
                    ivec = idxb[pl.ds(gg * 16, 16)]
                    rowv = (b * _BR + gg * 16) + iota16

                    @plsc.parallel_loop(0, _D // 2, unroll=8)
                    def _(c):
                        col = jnp.full((16,), 0, jnp.int32) + c
                        x = plsc.load_gather(cb_v, [ivec, col])
                        lo = plsc.bitcast(lax.shift_left(x, 16), jnp.float32)
                        hi = plsc.bitcast(
                            lax.bitwise_and(x, jnp.int32(-65536)), jnp.float32)
                        plsc.store_scatter(stg_v, [rowv, col], lo)
                        plsc.store_scatter(stg_v, [rowv, col + (_D // 2)], hi)

                    return c2

                lax.fori_loop(0, _BR // 16, grp, 0)

                out_desc(blk, b).start()

                @pl.when(blk + 2 < nblk)
                def _():
                    idx_start(blk + 2, b)
            return carry

        lax.fori_loop(0, nblk // 2, body, 0)
        out_desc(nblk - 2, 0).wait()
        out_desc(nblk - 1, 1).wait()

    return gather_kernel


def kernel(code_ids, codebook):
    b, k = code_ids.shape
    n = b * k
    info = plsc.get_sparse_core_info()
    nw = info.num_cores * info.num_subcores
    per_w = n // nw
    assert n % nw == 0 and per_w % _BR == 0 and (per_w // _BR) % 2 == 0, (n, nw)
    ids = code_ids.reshape(nw, per_w // _BR, _BR).astype(jnp.int32)
    cb_bf = codebook.astype(jnp.bfloat16)
    cb_pk = lax.bitcast_convert_type(
        jnp.stack([cb_bf[:, : _D // 2], cb_bf[:, _D // 2:]], axis=-1), jnp.int32)
    cb_pk = jnp.pad(cb_pk, ((0, 0), (0, _W - _D // 2)))
    out = _build(n, nw)(ids, cb_pk)
    return out.reshape(b, k, _D)


# speedup vs baseline: 1.6599x; 1.0975x over previous
import functools

import jax
import jax.numpy as jnp
from jax import lax
from jax.experimental import pallas as pl
from jax.experimental.pallas import tpu as pltpu
from jax.experimental.pallas import tpu_sc as plsc

_V = 1024
_D = 128
_W = _D // 2 + 1
_HB = 64
_SW = _D + 1


@functools.cache
def _build(n_total: int, nw: int):
    per_w = n_total // nw
    nsup = per_w // (2 * _HB)
    mesh = plsc.VectorSubcoreMesh(core_axis_name="c", subcore_axis_name="s")

    @functools.partial(
        pl.kernel,
        mesh=mesh,
        compiler_params=pltpu.CompilerParams(
            needs_layout_passes=False, disable_bounds_checks=True,
            use_tc_tiling_on_sc=False),
        out_type=jax.ShapeDtypeStruct((n_total, _D), jnp.float32),
        scratch_types=[
            pltpu.VMEM_SHARED((_V, _D), jnp.float32),
            pltpu.VMEM((_V, _W), jnp.int32),
            pltpu.VMEM((2, 2, _HB), jnp.int32),
            pltpu.VMEM((2, _HB, _D), jnp.float32),
            pltpu.VMEM((2 * _HB, _SW), jnp.float32),
            pltpu.SemaphoreType.DMA,
            pltpu.SemaphoreType.DMA,
            pltpu.SemaphoreType.DMA,
            pltpu.SemaphoreType.DMA,
        ],
    )
    def gather_kernel(ids_hbm, cb_hbm, cbpk_hbm, out_hbm, cb_sh, cb_v, idx_v,
                      sbuf, pstg, isem, gsem, os_sem, oc_sem):
        cid = lax.axis_index("c")
        sid = lax.axis_index("s")
        wid = sid * (nw // 16) + cid

        @pl.when(sid == 0)
        def _():
            pltpu.sync_copy(cb_hbm, cb_sh)

        pltpu.sync_copy(cbpk_hbm, cb_v)
        plsc.subcore_barrier()

        base = wid * per_w
        iota16 = lax.iota(jnp.int32, 16)

        def idx_start(j, b):
            return pltpu.async_copy(ids_hbm.at[wid, j], idx_v.at[b], isem)

        def idx_wait(j, b):
            pltpu.make_async_copy(ids_hbm.at[wid, j], idx_v.at[b], isem).wait()

        def g_desc(j, b):
            return pltpu.make_async_copy(
                cb_sh.at[idx_v.at[b, 0]], sbuf.at[b], gsem)

        def os_desc(j, b):
            return pltpu.make_async_copy(
                sbuf.at[b], out_hbm.at[pl.ds(base + j * 2 * _HB, _HB)], os_sem)

        def oc_desc(j, b):
            return pltpu.make_async_copy(
                pstg.at[pl.ds(b * _HB, _HB), pl.ds(0, _D)],
                out_hbm.at[pl.ds(base + j * 2 * _HB + _HB, _HB)], oc_sem)

        idx_start(0, 0)
        idx_start(1, 1)
        idx_wait(0, 0)
        g_desc(0, 0).start()

        def body(g, carry):
            for b in range(2):
                j = 2 * g + b
                g_desc(j, b).wait()
                os_desc(j, b).start()

                @pl.when(j >= 2)
                def _():
                    oc_desc(j - 2, b).wait()

                idxc = idx_v.at[b, 1]

                def grp(gg, c2):
                    ivec = idxc[pl.ds(gg * 16, 16)]
                    rowv = (b * _HB + gg * 16) + iota16

                    @plsc.parallel_loop(0, _D // 2, unroll=8)
                    def _(c):
                        col = jnp.full((16,), 0, jnp.int32) + c
                        x = plsc.load_gather(cb_v, [ivec, col])
                        lo = plsc.bitcast(lax.shift_left(x, 16), jnp.float32)
                        hi = plsc.bitcast(
                            lax.bitwise_and(x, jnp.int32(-65536)), jnp.float32)
                        plsc.store_scatter(pstg, [rowv, col], lo)
                        plsc.store_scatter(pstg, [rowv, col + (_D // 2)], hi)

                    return c2

                lax.fori_loop(0, _HB // 16, grp, 0)
                oc_desc(j, b).start()

                @pl.when(j >= 1)
                def _():
                    os_desc(j - 1, 1 - b).wait()

                @pl.when(j + 1 < nsup)
                def _():
                    idx_wait(j + 1, 1 - b)
                    g_desc(j + 1, 1 - b).start()

                @pl.when(j + 2 < nsup)
                def _():
                    idx_start(j + 2, b)
            return carry

        lax.fori_loop(0, nsup // 2, body, 0)
        os_desc(nsup - 1, 1).wait()
        oc_desc(nsup - 2, 0).wait()
        oc_desc(nsup - 1, 1).wait()

    return gather_kernel


def kernel(code_ids, codebook):
    b, k = code_ids.shape
    n = b * k
    info = plsc.get_sparse_core_info()
    nw = info.num_cores * info.num_subcores
    per_w = n // nw
    assert n % nw == 0 and per_w % (4 * _HB) == 0, (n, nw)
    ids = code_ids.reshape(nw, per_w // (2 * _HB), 2, _HB).astype(jnp.int32)
    cb_bf = codebook.astype(jnp.bfloat16)
    cb_pk = lax.bitcast_convert_type(
        jnp.stack([cb_bf[:, : _D // 2], cb_bf[:, _D // 2:]], axis=-1), jnp.int32)
    cb_pk = jnp.pad(cb_pk, ((0, 0), (0, _W - _D // 2)))
    out = _build(n, nw)(ids, codebook, cb_pk)
    return out.reshape(b, k, _D)
